# baseline (device time: 26578 ns/iter reference)
import jax
import jax.numpy as jnp
from jax import lax
from jax.experimental import pallas as pl
from jax.experimental.pallas import tpu as pltpu

N_DEV = 8


def kernel(x, w_mat):
    m_per, k = x.shape
    _, n_per = w_mat.shape

    def body(x_ref, w_ref, out_ref, xg_ref, send_sems, recv_sems):
        my = lax.axis_index("i")

        xg_ref[pl.ds(my * m_per, m_per), :] = x_ref[...].astype(jnp.bfloat16)

        sends = []
        for j in range(1, N_DEV):
            dst = lax.rem(my + j, N_DEV)
            rdma = pltpu.make_async_remote_copy(
                src_ref=xg_ref.at[pl.ds(my * m_per, m_per), :],
                dst_ref=xg_ref.at[pl.ds(my * m_per, m_per), :],
                send_sem=send_sems.at[j],
                recv_sem=recv_sems.at[my],
                device_id=(dst,),
                device_id_type=pl.DeviceIdType.MESH,
            )
            rdma.start()
            sends.append(rdma)

        for j in range(1, N_DEV):
            src = lax.rem(my + N_DEV - j, N_DEV)
            recv = pltpu.make_async_remote_copy(
                src_ref=xg_ref.at[pl.ds(src * m_per, m_per), :],
                dst_ref=xg_ref.at[pl.ds(src * m_per, m_per), :],
                send_sem=send_sems.at[0],
                recv_sem=recv_sems.at[src],
                device_id=(my,),
                device_id_type=pl.DeviceIdType.MESH,
            )
            recv.wait_recv()

        out_ref[...] = jnp.dot(
            xg_ref[...],
            w_ref[...].astype(jnp.bfloat16),
            preferred_element_type=jnp.float32,
        )

        for rdma in sends:
            rdma.wait_send()

    return pl.pallas_call(
        body,
        out_shape=jax.ShapeDtypeStruct((N_DEV * m_per, n_per), jnp.float32),
        in_specs=[
            pl.BlockSpec(memory_space=pltpu.VMEM),
            pl.BlockSpec(memory_space=pltpu.VMEM),
        ],
        out_specs=pl.BlockSpec(memory_space=pltpu.VMEM),
        scratch_shapes=[
            pltpu.VMEM((N_DEV * m_per, k), jnp.bfloat16),
            pltpu.SemaphoreType.DMA((N_DEV,)),
            pltpu.SemaphoreType.DMA((N_DEV,)),
        ],
    )(x, w_mat)


# device time: 25759 ns/iter; 1.0318x vs baseline; 1.0318x over previous
import jax
import jax.numpy as jnp
from jax import lax
from jax.experimental import pallas as pl
from jax.experimental.pallas import tpu as pltpu

N_DEV = 8
CW_HOPS = 4
CCW_HOPS = 3
S = 1


def kernel(x, w_mat):
    m_per, k = x.shape
    _, n_per = w_mat.shape
    kk = k // S

    def body(x_ref, w_ref, out_ref, xg_ref,
             cw_ssem, cw_rsem, ccw_ssem, ccw_rsem):
        my = lax.axis_index("i")
        right = lax.rem(my + 1, N_DEV)
        left = lax.rem(my + N_DEV - 1, N_DEV)

        barrier_sem = pltpu.get_barrier_semaphore()
        for nbr in (left, right):
            pl.semaphore_signal(
                barrier_sem, inc=1,
                device_id=(nbr,), device_id_type=pl.DeviceIdType.MESH,
            )
        pl.semaphore_wait(barrier_sem, 2)

        xg_ref[pl.ds(my * m_per, m_per), :] = x_ref[...].astype(jnp.bfloat16)

        def piece(org, p):
            return xg_ref.at[pl.ds(org * m_per, m_per), pl.ds(p * kk, kk)]

        def cw_send(h, p):
            org = lax.rem(my + N_DEV - h, N_DEV)
            rdma = pltpu.make_async_remote_copy(
                src_ref=piece(org, p), dst_ref=piece(org, p),
                send_sem=cw_ssem.at[h * S + p], recv_sem=cw_rsem.at[h * S + p],
                device_id=(right,), device_id_type=pl.DeviceIdType.MESH,
            )
            rdma.start()
            return rdma

        def ccw_send(h, p):
            org = lax.rem(my + h, N_DEV)
            rdma = pltpu.make_async_remote_copy(
                src_ref=piece(org, p), dst_ref=piece(org, p),
                send_sem=ccw_ssem.at[h * S + p], recv_sem=ccw_rsem.at[h * S + p],
                device_id=(left,), device_id_type=pl.DeviceIdType.MESH,
            )
            rdma.start()
            return rdma

        def wait_recv(direction, h, p):
            if direction == "cw":
                org = lax.rem(my + N_DEV - 1 - h, N_DEV)
                sem = cw_rsem.at[h * S + p]
            else:
                org = lax.rem(my + 1 + h, N_DEV)
                sem = ccw_rsem.at[h * S + p]
            recv = pltpu.make_async_remote_copy(
                src_ref=piece(org, p), dst_ref=piece(org, p),
                send_sem=cw_ssem.at[0], recv_sem=sem,
                device_id=(my,), device_id_type=pl.DeviceIdType.MESH,
            )
            recv.wait_recv()

        sends = []
        for p in range(S):
            sends.append(cw_send(0, p))
            sends.append(ccw_send(0, p))

        for h in range(CW_HOPS):
            for p in range(S):
                wait_recv("cw", h, p)
                if h + 1 < CW_HOPS:
                    sends.append(cw_send(h + 1, p))
            if h < CCW_HOPS:
                for p in range(S):
                    wait_recv("ccw", h, p)
                    if h + 1 < CCW_HOPS:
                        sends.append(ccw_send(h + 1, p))

        out_ref[...] = jnp.dot(
            xg_ref[...],
            w_ref[...].astype(jnp.bfloat16),
            preferred_element_type=jnp.float32,
        )

        for rdma in sends:
            rdma.wait_send()

    return pl.pallas_call(
        body,
        out_shape=jax.ShapeDtypeStruct((N_DEV * m_per, n_per), jnp.float32),
        in_specs=[
            pl.BlockSpec(memory_space=pltpu.VMEM),
            pl.BlockSpec(memory_space=pltpu.VMEM),
        ],
        out_specs=pl.BlockSpec(memory_space=pltpu.VMEM),
        scratch_shapes=[
            pltpu.VMEM((N_DEV * m_per, k), jnp.bfloat16),
            pltpu.SemaphoreType.DMA((CW_HOPS * S,)),
            pltpu.SemaphoreType.DMA((CW_HOPS * S,)),
            pltpu.SemaphoreType.DMA((CCW_HOPS * S,)),
            pltpu.SemaphoreType.DMA((CCW_HOPS * S,)),
        ],
        compiler_params=pltpu.CompilerParams(collective_id=0),
    )(x, w_mat)


# device time: 23129 ns/iter; 1.1491x vs baseline; 1.1137x over previous
import jax
import jax.numpy as jnp
from jax import lax
from jax.experimental import pallas as pl
from jax.experimental.pallas import tpu as pltpu

N_DEV = 8
CW_HOPS = 4
CCW_HOPS = 3
S = 4


def kernel(x, w_mat):
    m_per, k = x.shape
    _, n_per = w_mat.shape
    kk = k // S

    def body(x_ref, w_ref, out_ref, xg_ref,
             cw_ssem, cw_rsem, ccw_ssem, ccw_rsem):
        my = lax.axis_index("i")
        right = lax.rem(my + 1, N_DEV)
        left = lax.rem(my + N_DEV - 1, N_DEV)

        barrier_sem = pltpu.get_barrier_semaphore()
        for nbr in (left, right):
            pl.semaphore_signal(
                barrier_sem, inc=1,
                device_id=(nbr,), device_id_type=pl.DeviceIdType.MESH,
            )
        pl.semaphore_wait(barrier_sem, 2)

        xg_ref[pl.ds(my * m_per, m_per), :] = x_ref[...].astype(jnp.bfloat16)

        def piece(org, p):
            return xg_ref.at[pl.ds(org * m_per, m_per), pl.ds(p * kk, kk)]

        def cw_send(h, p):
            org = lax.rem(my + N_DEV - h, N_DEV)
            rdma = pltpu.make_async_remote_copy(
                src_ref=piece(org, p), dst_ref=piece(org, p),
                send_sem=cw_ssem.at[h * S + p], recv_sem=cw_rsem.at[h * S + p],
                device_id=(right,), device_id_type=pl.DeviceIdType.MESH,
            )
            rdma.start()
            return rdma

        def ccw_send(h, p):
            org = lax.rem(my + h, N_DEV)
            rdma = pltpu.make_async_remote_copy(
                src_ref=piece(org, p), dst_ref=piece(org, p),
                send_sem=ccw_ssem.at[h * S + p], recv_sem=ccw_rsem.at[h * S + p],
                device_id=(left,), device_id_type=pl.DeviceIdType.MESH,
            )
            rdma.start()
            return rdma

        def wait_recv(direction, h, p):
            if direction == "cw":
                org = lax.rem(my + N_DEV - 1 - h, N_DEV)
                sem = cw_rsem.at[h * S + p]
            else:
                org = lax.rem(my + 1 + h, N_DEV)
                sem = ccw_rsem.at[h * S + p]
            recv = pltpu.make_async_remote_copy(
                src_ref=piece(org, p), dst_ref=piece(org, p),
                send_sem=cw_ssem.at[0], recv_sem=sem,
                device_id=(my,), device_id_type=pl.DeviceIdType.MESH,
            )
            recv.wait_recv()

        sends = []
        for p in range(S):
            sends.append(cw_send(0, p))
            sends.append(ccw_send(0, p))

        for h in range(CW_HOPS):
            for p in range(S):
                wait_recv("cw", h, p)
                if h + 1 < CW_HOPS:
                    sends.append(cw_send(h + 1, p))
            if h < CCW_HOPS:
                for p in range(S):
                    wait_recv("ccw", h, p)
                    if h + 1 < CCW_HOPS:
                        sends.append(ccw_send(h + 1, p))

        out_ref[...] = jnp.dot(
            xg_ref[...],
            w_ref[...].astype(jnp.bfloat16),
            preferred_element_type=jnp.float32,
        )

        for rdma in sends:
            rdma.wait_send()

    return pl.pallas_call(
        body,
        out_shape=jax.ShapeDtypeStruct((N_DEV * m_per, n_per), jnp.float32),
        in_specs=[
            pl.BlockSpec(memory_space=pltpu.VMEM),
            pl.BlockSpec(memory_space=pltpu.VMEM),
        ],
        out_specs=pl.BlockSpec(memory_space=pltpu.VMEM),
        scratch_shapes=[
            pltpu.VMEM((N_DEV * m_per, k), jnp.bfloat16),
            pltpu.SemaphoreType.DMA((CW_HOPS * S,)),
            pltpu.SemaphoreType.DMA((CW_HOPS * S,)),
            pltpu.SemaphoreType.DMA((CCW_HOPS * S,)),
            pltpu.SemaphoreType.DMA((CCW_HOPS * S,)),
        ],
        compiler_params=pltpu.CompilerParams(collective_id=0),
    )(x, w_mat)


# device time: 21614 ns/iter; 1.2297x vs baseline; 1.0701x over previous
import jax
import jax.numpy as jnp
from jax import lax
from jax.experimental import pallas as pl
from jax.experimental.pallas import tpu as pltpu

N_DEV = 8
HOPS = 4
S = 4


def _pieces(h):
    if h < HOPS - 1:
        return range(S)
    return range(S // 2)


def _pieces_ccw(h):
    if h < HOPS - 1:
        return range(S)
    return range(S // 2, S)


def kernel(x, w_mat):
    m_per, k = x.shape
    _, n_per = w_mat.shape
    kk = k // S

    def body(x_ref, w_ref, out_ref, xg_ref, w_bf,
             cw_ssem, cw_rsem, ccw_ssem, ccw_rsem):
        my = lax.axis_index("i")
        right = lax.rem(my + 1, N_DEV)
        left = lax.rem(my + N_DEV - 1, N_DEV)

        barrier_sem = pltpu.get_barrier_semaphore()
        for nbr in (left, right):
            pl.semaphore_signal(
                barrier_sem, inc=1,
                device_id=(nbr,), device_id_type=pl.DeviceIdType.MESH,
            )
        xg_ref[pl.ds(my * m_per, m_per), :] = x_ref[...].astype(jnp.bfloat16)
        pl.semaphore_wait(barrier_sem, 2)

        def piece(org, p):
            return xg_ref.at[pl.ds(org * m_per, m_per), pl.ds(p * kk, kk)]

        sends = []

        def cw_send(h, p):
            org = lax.rem(my + N_DEV - h, N_DEV)
            rdma = pltpu.make_async_remote_copy(
                src_ref=piece(org, p), dst_ref=piece(org, p),
                send_sem=cw_ssem.at[h * S + p], recv_sem=cw_rsem.at[h * S + p],
                device_id=(right,), device_id_type=pl.DeviceIdType.MESH,
            )
            rdma.start()
            sends.append(rdma)

        def ccw_send(h, p):
            org = lax.rem(my + h, N_DEV)
            rdma = pltpu.make_async_remote_copy(
                src_ref=piece(org, p), dst_ref=piece(org, p),
                send_sem=ccw_ssem.at[h * S + p], recv_sem=ccw_rsem.at[h * S + p],
                device_id=(left,), device_id_type=pl.DeviceIdType.MESH,
            )
            rdma.start()
            sends.append(rdma)

        def wait_recv_cw(h, p):
            org = lax.rem(my + N_DEV - 1 - h, N_DEV)
            recv = pltpu.make_async_remote_copy(
                src_ref=piece(org, p), dst_ref=piece(org, p),
                send_sem=cw_ssem.at[0], recv_sem=cw_rsem.at[h * S + p],
                device_id=(my,), device_id_type=pl.DeviceIdType.MESH,
            )
            recv.wait_recv()

        def wait_recv_ccw(h, p):
            org = lax.rem(my + 1 + h, N_DEV)
            recv = pltpu.make_async_remote_copy(
                src_ref=piece(org, p), dst_ref=piece(org, p),
                send_sem=ccw_ssem.at[0], recv_sem=ccw_rsem.at[h * S + p],
                device_id=(my,), device_id_type=pl.DeviceIdType.MESH,
            )
            recv.wait_recv()

        def chunk_gemm(org):
            out_ref[pl.ds(org * m_per, m_per), :] = jnp.dot(
                xg_ref[pl.ds(org * m_per, m_per), :], w_bf[...],
                preferred_element_type=jnp.float32,
            )

        for p in _pieces(0):
            cw_send(0, p)
        for p in _pieces_ccw(0):
            ccw_send(0, p)

        w_bf[...] = w_ref[...].astype(jnp.bfloat16)
        chunk_gemm(my)

        for h in range(HOPS):
            for p in _pieces(h):
                wait_recv_cw(h, p)
                if h + 1 < HOPS and p in _pieces(h + 1):
                    cw_send(h + 1, p)
            for p in _pieces_ccw(h):
                wait_recv_ccw(h, p)
                if h + 1 < HOPS and p in _pieces_ccw(h + 1):
                    ccw_send(h + 1, p)
            if h < HOPS - 1:
                chunk_gemm(lax.rem(my + N_DEV - 1 - h, N_DEV))
                chunk_gemm(lax.rem(my + 1 + h, N_DEV))
            else:
                chunk_gemm(lax.rem(my + HOPS, N_DEV))

        for rdma in sends:
            rdma.wait_send()

    return pl.pallas_call(
        body,
        out_shape=jax.ShapeDtypeStruct((N_DEV * m_per, n_per), jnp.float32),
        in_specs=[
            pl.BlockSpec(memory_space=pltpu.VMEM),
            pl.BlockSpec(memory_space=pltpu.VMEM),
        ],
        out_specs=pl.BlockSpec(memory_space=pltpu.VMEM),
        scratch_shapes=[
            pltpu.VMEM((N_DEV * m_per, k), jnp.bfloat16),
            pltpu.VMEM((k, n_per), jnp.bfloat16),
            pltpu.SemaphoreType.DMA((HOPS * S,)),
            pltpu.SemaphoreType.DMA((HOPS * S,)),
            pltpu.SemaphoreType.DMA((HOPS * S,)),
            pltpu.SemaphoreType.DMA((HOPS * S,)),
        ],
        compiler_params=pltpu.CompilerParams(collective_id=0),
    )(x, w_mat)


# device time: 19211 ns/iter; 1.3835x vs baseline; 1.1251x over previous
import jax
import jax.numpy as jnp
from jax import lax
from jax.experimental import pallas as pl
from jax.experimental.pallas import tpu as pltpu

N_DEV = 8


def kernel(x, w_mat):
    m_per, k = x.shape
    _, n_per = w_mat.shape
    kh = k // 2

    def body(x_ref, w_ref, out_ref, xg_ref, w_bf,
             z_s, z_r, cw_s, cw_r, ccw_s, ccw_r):
        my = lax.axis_index("i")
        q = lax.rem(my, 4)
        zbase = my - q
        right = zbase + lax.rem(q + 1, 4)
        left = zbase + lax.rem(q + 3, 4)
        o2 = zbase + lax.rem(q + 2, 4)
        partner = lax.rem(my + 4, N_DEV)
        leftp = lax.rem(left + 4, N_DEV)
        rightp = lax.rem(right + 4, N_DEV)
        o2p = lax.rem(o2 + 4, N_DEV)

        barrier_sem = pltpu.get_barrier_semaphore()
        for nbr in (left, right, partner):
            pl.semaphore_signal(
                barrier_sem, inc=1,
                device_id=(nbr,), device_id_type=pl.DeviceIdType.MESH,
            )
        xg_ref[pl.ds(my * m_per, m_per), :] = x_ref[...].astype(jnp.bfloat16)
        pl.semaphore_wait(barrier_sem, 3)

        def full(org):
            return xg_ref.at[pl.ds(org * m_per, m_per), :]

        def half(org, h):
            return xg_ref.at[pl.ds(org * m_per, m_per), pl.ds(h * kh, kh)]

        sends = []

        def send(src, dst_dev, ssem, rsem):
            rdma = pltpu.make_async_remote_copy(
                src_ref=src, dst_ref=src,
                send_sem=ssem, recv_sem=rsem,
                device_id=(dst_dev,), device_id_type=pl.DeviceIdType.MESH,
            )
            rdma.start()
            sends.append(rdma)

        def wait_recv(dst, rsem):
            recv = pltpu.make_async_remote_copy(
                src_ref=dst, dst_ref=dst,
                send_sem=z_s.at[0], recv_sem=rsem,
                device_id=(my,), device_id_type=pl.DeviceIdType.MESH,
            )
            recv.wait_recv()

        def chunk_gemm(org):
            out_ref[pl.ds(org * m_per, m_per), :] = jnp.dot(
                xg_ref[pl.ds(org * m_per, m_per), :], w_bf[...],
                preferred_element_type=jnp.float32,
            )

        send(full(my), partner, z_s.at[0], z_r.at[0])
        send(full(my), right, cw_s.at[0], cw_r.at[0])
        send(full(my), left, ccw_s.at[0], ccw_r.at[0])

        w_bf[...] = w_ref[...].astype(jnp.bfloat16)
        chunk_gemm(my)

        wait_recv(full(partner), z_r.at[0])
        send(full(partner), right, cw_s.at[1], cw_r.at[1])
        send(full(partner), left, ccw_s.at[1], ccw_r.at[1])
        chunk_gemm(partner)

        wait_recv(full(left), cw_r.at[0])
        send(half(left, 0), right, cw_s.at[2], cw_r.at[2])
        chunk_gemm(left)

        wait_recv(full(right), ccw_r.at[0])
        send(half(right, 1), left, ccw_s.at[2], ccw_r.at[2])
        chunk_gemm(right)

        wait_recv(full(leftp), cw_r.at[1])
        send(half(leftp, 0), right, cw_s.at[3], cw_r.at[3])
        chunk_gemm(leftp)

        wait_recv(full(rightp), ccw_r.at[1])
        send(half(rightp, 1), left, ccw_s.at[3], ccw_r.at[3])
        chunk_gemm(rightp)

        wait_recv(half(o2, 0), cw_r.at[2])
        wait_recv(half(o2, 1), ccw_r.at[2])
        chunk_gemm(o2)
        wait_recv(half(o2p, 0), cw_r.at[3])
        wait_recv(half(o2p, 1), ccw_r.at[3])
        chunk_gemm(o2p)

        for rdma in sends:
            rdma.wait_send()

    return pl.pallas_call(
        body,
        out_shape=jax.ShapeDtypeStruct((N_DEV * m_per, n_per), jnp.float32),
        in_specs=[
            pl.BlockSpec(memory_space=pltpu.VMEM),
            pl.BlockSpec(memory_space=pltpu.VMEM),
        ],
        out_specs=pl.BlockSpec(memory_space=pltpu.VMEM),
        scratch_shapes=[
            pltpu.VMEM((N_DEV * m_per, k), jnp.bfloat16),
            pltpu.VMEM((k, n_per), jnp.bfloat16),
            pltpu.SemaphoreType.DMA((1,)),
            pltpu.SemaphoreType.DMA((1,)),
            pltpu.SemaphoreType.DMA((4,)),
            pltpu.SemaphoreType.DMA((4,)),
            pltpu.SemaphoreType.DMA((4,)),
            pltpu.SemaphoreType.DMA((4,)),
        ],
        compiler_params=pltpu.CompilerParams(collective_id=0),
    )(x, w_mat)
